# Initial kernel scaffold; baseline (speedup 1.0000x reference)
import jax
import jax.numpy as jnp
from jax.experimental import pallas as pl


def kernel(image_resnet, params, A, ref_vertices):
    raise NotImplementedError("write your pallas kernel here")


# speedup vs baseline: 2.1834x; 2.1834x over previous
import functools

import jax
import jax.numpy as jnp
from jax import lax
from jax.experimental import pallas as pl

_NPAD = 1792
_F32 = jnp.float32


def _row(v):
    return v.reshape(1, -1)


def _mm(a, b):
    return lax.dot_general(a, b, (((1,), (0,)), ((), ())),
                           preferred_element_type=_F32)


def _grp(v, gmats):
    ch = v.shape[1]
    if ch in gmats:
        return _mm(v, gmats[ch][...])
    big = max(gmats)
    parts = [_mm(v[:, i:i + big], gmats[big][...]) for i in range(0, ch, big)]
    return jnp.concatenate(parts, axis=1)


def _gn_relu(x, g, b, gmats, valid, inv_cnt):
    xm = jnp.where(valid, x, 0.0)
    mean = _grp(jnp.sum(xm, axis=0, keepdims=True), gmats) * inv_cnt
    d = x - mean
    dm = jnp.where(valid, d, 0.0)
    var = _grp(jnp.sum(dm * dm, axis=0, keepdims=True), gmats) * inv_cnt
    y = (d * lax.rsqrt(var + 1e-5)) * g + b
    return jnp.maximum(y, 0.0)


def _net_body(n, bsz, skip_flags, gsizes, *refs):
    shape_ref, cam_ref = refs[-2], refs[-1]
    it = iter(refs[:-2])
    image_ref = next(it)
    rvt_ref = next(it)
    a_ref = next(it)
    wrvt_ref = next(it)
    wimgt_ref = next(it)
    gl0b_ref = next(it)
    blocks = []
    for sk in skip_flags:
        blocks.append([next(it) for _ in range(14 if sk else 12)])
    (sh1Wt, sh1b, sh2Wt, sh2b, shg, shb, sh3Wt, sh3b,
     cgng, cgnb, cglWt, cglb, clinWt, clinb) = [next(it) for _ in range(14)]
    gmats = {sz: next(it) for sz in gsizes}

    valid = lax.broadcasted_iota(jnp.int32, (_NPAD, 1), 0) < n
    inv_cnt = 1.0 / (8.0 * n)

    def gn(v, g, b):
        return _gn_relu(v, g[...], b[...], gmats, valid, inv_cnt)

    def one_batch(b, carry):
        img_row = image_ref[pl.ds(b, 1), :]
        x = _mm(rvt_ref[...], wrvt_ref[...])
        x = x + _mm(img_row, wimgt_ref[...]) + gl0b_ref[...]
        for sk, blk in zip(skip_flags, blocks):
            if sk:
                xs = _mm(x, blk[12][...]) + blk[13][...]
            y = gn(x, blk[0], blk[1])
            y = _mm(y, blk[2][...]) + blk[3][...]
            y = gn(y, blk[4], blk[5])
            s = _mm(y, blk[6][...])
            z = _mm(a_ref[...], s) + blk[7][...]
            z = gn(z, blk[8], blk[9])
            z = _mm(z, blk[10][...]) + blk[11][...]
            x = (xs if sk else x) + z
        t = jnp.maximum(_mm(x, sh1Wt[...]) + sh1b[...], 0.0)
        t = _mm(t, sh2Wt[...]) + sh2b[...]
        t = gn(t, shg, shb)
        so = _mm(t, sh3Wt[...]) + sh3b[...]
        shape_ref[pl.ds(b, 1), :, :] = so.reshape(1, _NPAD, 8)
        c = gn(x, cgng, cgnb)
        c = jnp.maximum(_mm(c, cglWt[...]) + cglb[...], 0.0)
        q = c[:, 0:1] * clinWt[...]
        cam_ref[pl.ds(b, 1), :] = jnp.sum(q, axis=0, keepdims=True) + clinb[...]
        return carry

    lax.fori_loop(0, bsz, one_batch, 0)


def kernel(image_resnet, params, A, ref_vertices):
    bsz = image_resnet.shape[0]
    n = A.shape[0]
    gl0W = params['gl0_W']
    ops = [
        image_resnet.astype(_F32),
        jnp.pad(ref_vertices.astype(_F32).T, ((0, _NPAD - n), (0, 5))),
        jnp.pad(A.astype(_F32), ((0, _NPAD - n), (0, _NPAD - n))),
        jnp.pad(gl0W[:, :3].T, ((0, 5), (0, 0))),
        gl0W[:, 3:].T,
        _row(params['gl0_b']),
    ]
    skip_flags = []
    for p in params['blocks']:
        skip_flags.append('skip_W' in p)
        ops += [_row(p['pre_g']), _row(p['pre_b']), p['lin1_W'].T, _row(p['lin1_b']),
                _row(p['n1_g']), _row(p['n1_b']), p['conv_W'], _row(p['conv_b']),
                _row(p['n2_g']), _row(p['n2_b']), p['lin2_W'].T, _row(p['lin2_b'])]
        if skip_flags[-1]:
            ops += [p['skip_W'].T, _row(p['skip_b'])]
    ops += [params['sh1_W'].T, _row(params['sh1_b']),
            params['sh2_W'].T, _row(params['sh2_b']),
            _row(params['sh_gn_g']), _row(params['sh_gn_b']),
            jnp.pad(params['sh3_W'].T, ((0, 0), (0, 5))),
            jnp.pad(_row(params['sh3_b']), ((0, 0), (0, 5))),
            _row(params['cam_gn_g']), _row(params['cam_gn_b']),
            jnp.pad(params['cam_gl_W'].T, ((0, 0), (0, 7))),
            params['cam_gl_b'].reshape(1, 1),
            jnp.pad(params['cam_lin_W'].T, ((0, _NPAD - n), (0, 5))),
            jnp.pad(_row(params['cam_lin_b']), ((0, 0), (0, 5)))]
    gsizes = tuple(sorted(
        {min(s, 512) for s in
         {p['pre_g'].shape[0] for p in params['blocks']}
         | {p['n1_g'].shape[0] for p in params['blocks']}
         | {params['sh_gn_g'].shape[0], params['cam_gn_g'].shape[0]}},
        reverse=True))
    for sz in gsizes:
        ops.append(jnp.kron(jnp.eye(sz // 8, dtype=_F32),
                            jnp.ones((8, 8), _F32)))
    body = functools.partial(_net_body, n, bsz, tuple(skip_flags), gsizes)
    shape_pad, cam_pad = pl.pallas_call(
        body,
        out_shape=(jax.ShapeDtypeStruct((bsz, _NPAD, 8), _F32),
                   jax.ShapeDtypeStruct((bsz, 8), _F32)),
    )(*ops)
    return (jnp.swapaxes(shape_pad[:, :n, :3], 1, 2), cam_pad[:, :3])


# speedup vs baseline: 2.4039x; 1.1010x over previous
import functools

import jax
import jax.numpy as jnp
from jax import lax
from jax.experimental import pallas as pl

_NPAD = 1792
_F32 = jnp.float32


def _row(v):
    return v.reshape(1, -1)


def _mm(a, b):
    return lax.dot_general(a, b, (((1,), (0,)), ((), ())),
                           preferred_element_type=_F32)


def _grp(v, gmats):
    ch = v.shape[1]
    if ch in gmats:
        return _mm(v, gmats[ch][...])
    big = max(gmats)
    parts = [_mm(v[:, i:i + big], gmats[big][...]) for i in range(0, ch, big)]
    return jnp.concatenate(parts, axis=1)


def _gn_relu(x, g, b, gmats, valid, inv_cnt, pad_rows):
    xm = jnp.where(valid, x, 0.0)
    mean = _grp(jnp.sum(xm, axis=0, keepdims=True), gmats) * inv_cnt
    d = xm - mean
    var = (_grp(jnp.sum(d * d, axis=0, keepdims=True), gmats) * inv_cnt
           - (8.0 * pad_rows * inv_cnt) * mean * mean)
    y = (d * lax.rsqrt(var + 1e-5)) * g + b
    return jnp.maximum(y, 0.0)


def _net_body(n, bsz, skip_flags, gsizes, *refs):
    shape_ref, cam_ref = refs[-2], refs[-1]
    it = iter(refs[:-2])
    image_ref = next(it)
    rvt_ref = next(it)
    a_ref = next(it)
    wrvt_ref = next(it)
    wimgt_ref = next(it)
    gl0b_ref = next(it)
    blocks = []
    for sk in skip_flags:
        blocks.append([next(it) for _ in range(14 if sk else 12)])
    (sh1Wt, sh1b, sh2Wt, sh2b, shg, shb, sh3Wt, sh3b,
     cgng, cgnb, cglWt, cglb, clinWt, clinb) = [next(it) for _ in range(14)]
    gmats = {sz: next(it) for sz in gsizes}

    valid = lax.broadcasted_iota(jnp.int32, (_NPAD, 1), 0) < n
    inv_cnt = 1.0 / (8.0 * n)

    def gn(v, g, b):
        return _gn_relu(v, g[...], b[...], gmats, valid, inv_cnt,
                        float(_NPAD - n))

    def one_batch(b, carry):
        img_row = image_ref[pl.ds(b, 1), :]
        x = _mm(rvt_ref[...], wrvt_ref[...])
        x = x + _mm(img_row, wimgt_ref[...]) + gl0b_ref[...]
        for sk, blk in zip(skip_flags, blocks):
            if sk:
                xs = _mm(x, blk[12][...]) + blk[13][...]
            y = gn(x, blk[0], blk[1])
            y = _mm(y, blk[2][...]) + blk[3][...]
            y = gn(y, blk[4], blk[5])
            s = _mm(y, blk[6][...])
            z = _mm(a_ref[...], s.astype(jnp.bfloat16)) + blk[7][...]
            z = gn(z, blk[8], blk[9])
            z = _mm(z, blk[10][...]) + blk[11][...]
            x = (xs if sk else x) + z
        t = jnp.maximum(_mm(x, sh1Wt[...]) + sh1b[...], 0.0)
        t = _mm(t, sh2Wt[...]) + sh2b[...]
        t = gn(t, shg, shb)
        so = _mm(t, sh3Wt[...]) + sh3b[...]
        shape_ref[pl.ds(b, 1), :, :] = so.reshape(1, _NPAD, 8)
        c = gn(x, cgng, cgnb)
        c = jnp.maximum(_mm(c, cglWt[...]) + cglb[...], 0.0)
        q = c[:, 0:1] * clinWt[...]
        cam_ref[pl.ds(b, 1), :] = jnp.sum(q, axis=0, keepdims=True) + clinb[...]
        return carry

    lax.fori_loop(0, bsz, one_batch, 0)


def kernel(image_resnet, params, A, ref_vertices):
    bsz = image_resnet.shape[0]
    n = A.shape[0]
    gl0W = params['gl0_W']
    ops = [
        image_resnet.astype(_F32),
        jnp.pad(ref_vertices.astype(_F32).T, ((0, _NPAD - n), (0, 5))),
        jnp.pad(A.astype(jnp.bfloat16), ((0, _NPAD - n), (0, _NPAD - n))),
        jnp.pad(gl0W[:, :3].T, ((0, 5), (0, 0))),
        gl0W[:, 3:].T,
        _row(params['gl0_b']),
    ]
    skip_flags = []
    for p in params['blocks']:
        skip_flags.append('skip_W' in p)
        ops += [_row(p['pre_g']), _row(p['pre_b']), p['lin1_W'].T, _row(p['lin1_b']),
                _row(p['n1_g']), _row(p['n1_b']), p['conv_W'], _row(p['conv_b']),
                _row(p['n2_g']), _row(p['n2_b']), p['lin2_W'].T, _row(p['lin2_b'])]
        if skip_flags[-1]:
            ops += [p['skip_W'].T, _row(p['skip_b'])]
    ops += [params['sh1_W'].T, _row(params['sh1_b']),
            params['sh2_W'].T, _row(params['sh2_b']),
            _row(params['sh_gn_g']), _row(params['sh_gn_b']),
            jnp.pad(params['sh3_W'].T, ((0, 0), (0, 5))),
            jnp.pad(_row(params['sh3_b']), ((0, 0), (0, 5))),
            _row(params['cam_gn_g']), _row(params['cam_gn_b']),
            jnp.pad(params['cam_gl_W'].T, ((0, 0), (0, 7))),
            params['cam_gl_b'].reshape(1, 1),
            jnp.pad(params['cam_lin_W'].T, ((0, _NPAD - n), (0, 5))),
            jnp.pad(_row(params['cam_lin_b']), ((0, 0), (0, 5)))]
    gsizes = tuple(sorted(
        {min(s, 512) for s in
         {p['pre_g'].shape[0] for p in params['blocks']}
         | {p['n1_g'].shape[0] for p in params['blocks']}
         | {params['sh_gn_g'].shape[0], params['cam_gn_g'].shape[0]}},
        reverse=True))
    for sz in gsizes:
        ops.append(jnp.kron(jnp.eye(sz // 8, dtype=_F32),
                            jnp.ones((8, 8), _F32)))
    body = functools.partial(_net_body, n, bsz, tuple(skip_flags), gsizes)
    shape_pad, cam_pad = pl.pallas_call(
        body,
        out_shape=(jax.ShapeDtypeStruct((bsz, _NPAD, 8), _F32),
                   jax.ShapeDtypeStruct((bsz, 8), _F32)),
    )(*ops)
    return (jnp.swapaxes(shape_pad[:, :n, :3], 1, 2), cam_pad[:, :3])
